# baseline (speedup 1.0000x reference)
import functools

import jax
import jax.numpy as jnp
from jax import lax
from jax.experimental import pallas as pl
from jax.experimental.pallas import tpu as pltpu
from jax.experimental.pallas import tpu_sc as plsc

_N = 10000
_D = 128
_E = 320000

_NC = 2
_NS = 16
_NW = _NC * _NS
_EPW = _E // _NW
_CH = 96
_NCHUNK = 105
_EPWP = _NCHUNK * _CH
_NP = 10240
_RPT = _NP // _NS
_ZR = 8


def _sc_agg(h, src, dst):
  mesh = plsc.VectorSubcoreMesh(core_axis_name="c", subcore_axis_name="s")

  @functools.partial(
      pl.kernel,
      out_type=(
          jax.ShapeDtypeStruct((_NC, _NP, _D), jnp.float32),
          jax.ShapeDtypeStruct((_NC, _NP, _D), jnp.float32),
      ),
      mesh=mesh,
      scratch_types=[
          pltpu.VMEM((_CH,), jnp.int32),
          pltpu.VMEM((_CH,), jnp.int32),
          pltpu.VMEM((_CH,), jnp.int32),
          pltpu.VMEM((_CH,), jnp.int32),
          pltpu.VMEM((_NCHUNK, _CH), jnp.int32),
          pltpu.VMEM((_CH, _D), jnp.float32),
          pltpu.VMEM((_CH, _D), jnp.float32),
          pltpu.VMEM((_ZR, _D), jnp.float32),
          pltpu.VMEM_SHARED((_NP, _D), jnp.float32),
          pltpu.SemaphoreType.DMA,
          pltpu.SemaphoreType.DMA,
          pltpu.SemaphoreType.DMA,
          pltpu.SemaphoreType.DMA,
          pltpu.SemaphoreType.DMA,
          pltpu.SemaphoreType.DMA,
          pltpu.SemaphoreType.DMA,
          pltpu.SemaphoreType.DMA,
      ],
  )
  def k(h_hbm, src_hbm, dst_hbm, agg_out, cnt_out,
        sb0, sb1, sb2, sb3, didx_v, rows0_v, rows1_v, zd_v, acc_sp,
        g0, g1, si0, si1, si2, si3, sem_s, sem_z):
    cid = lax.axis_index("c")
    sid = lax.axis_index("s")
    wid = sid * _NC + cid
    sbuf = [sb0, sb1, sb2, sb3]
    sisem = [si0, si1, si2, si3]
    rows = [rows0_v, rows1_v]
    gsem = [g0, g1]

    zero16 = jnp.zeros((16,), jnp.float32)
    one16 = jnp.ones((16,), jnp.float32)

    pltpu.sync_copy(dst_hbm.at[wid], didx_v)

    def fill_zd(t, _):
      zd_v[t // (_D // 16), pl.ds((t % (_D // 16)) * 16, 16)] = zero16
      return 0
    lax.fori_loop(0, _ZR * (_D // 16), fill_zd, 0)

    def zero_own_rows(_unused):
      def zero_slab(z, _):
        r0 = sid * _RPT + z * _ZR
        pltpu.async_copy(zd_v, acc_sp.at[pl.ds(r0, _ZR), :], sem_z)
        return 0
      lax.fori_loop(0, _RPT // _ZR, zero_slab, 0)

      def zero_drain(z, _):
        r0 = sid * _RPT + z * _ZR
        pltpu.make_async_copy(zd_v, acc_sp.at[pl.ds(r0, _ZR), :],
                              sem_z).wait()
        return 0
      lax.fori_loop(0, _RPT // _ZR, zero_drain, 0)

    zero_own_rows(None)
    plsc.subcore_barrier()

    for j in range(4):
      pltpu.async_copy(src_hbm.at[wid, j], sbuf[j], sisem[j])
    pltpu.make_async_copy(src_hbm.at[wid, 0], sbuf[0], sisem[0]).wait()
    pltpu.async_copy(h_hbm.at[sbuf[0]], rows0_v, g0)
    pltpu.make_async_copy(src_hbm.at[wid, 1], sbuf[1], sisem[1]).wait()
    pltpu.async_copy(h_hbm.at[sbuf[1]], rows1_v, g1)

    def quad(gidx, _):
      for b in range(4):
        i = 4 * gidx + b
        r = rows[b % 2]
        pltpu.make_async_copy(h_hbm.at[sbuf[b]], r, gsem[b % 2]).wait()

        @pl.when(i + 4 < _NCHUNK)
        def _():
          pltpu.async_copy(src_hbm.at[wid, i + 4], sbuf[b], sisem[b])
        pltpu.sync_copy(r, acc_sp.at[didx_v.at[i]], add=True)

        @pl.when(i + 2 < _NCHUNK)
        def _():
          b2 = (b + 2) % 4
          pltpu.make_async_copy(src_hbm.at[wid, i + 2], sbuf[b2],
                                sisem[b2]).wait()
          pltpu.async_copy(h_hbm.at[sbuf[b2]], r, gsem[b % 2])
      return 0
    lax.fori_loop(0, (_NCHUNK - 1) // 4, quad, 0)

    pltpu.make_async_copy(h_hbm.at[sbuf[0]], rows0_v, g0).wait()
    pltpu.sync_copy(rows0_v, acc_sp.at[didx_v.at[_NCHUNK - 1]], add=True)

    plsc.subcore_barrier()

    r0 = sid * _RPT
    pltpu.sync_copy(acc_sp.at[pl.ds(r0, _RPT), :],
                    agg_out.at[cid, pl.ds(r0, _RPT), :])

    def fill_ones(t, _):
      rows0_v[t // (_D // 16), pl.ds((t % (_D // 16)) * 16, 16)] = one16
      return 0
    lax.fori_loop(0, _CH * (_D // 16), fill_ones, 0)
    zero_own_rows(None)
    plsc.subcore_barrier()

    def cnt_fire(i, _):
      pltpu.async_copy(rows0_v, acc_sp.at[didx_v.at[i]], sem_s, add=True)
      return 0
    lax.fori_loop(0, _NCHUNK, cnt_fire, 0)

    def cnt_drain(i, _):
      pltpu.make_async_copy(rows0_v, acc_sp.at[didx_v.at[i]], sem_s).wait()
      return 0
    lax.fori_loop(0, _NCHUNK, cnt_drain, 0)

    plsc.subcore_barrier()

    pltpu.sync_copy(acc_sp.at[pl.ds(r0, _RPT), :],
                    cnt_out.at[cid, pl.ds(r0, _RPT), :])

  return k(h, src, dst)


_BM = 1000


def _gelu(y):
  return 0.5 * y * (1.0 + lax.erf(y * 0.7071067811865476))


def _tc_layer1_body(agg_ref, cnt_ref, h_ref, wl_ref, wr_ref, b_ref, o_ref):
  agg = agg_ref[0] + agg_ref[1]
  cnt = cnt_ref[0, :, 0:1] + cnt_ref[1, :, 0:1]
  mean = agg / jnp.maximum(cnt, 1.0)
  y = (jnp.dot(mean, wl_ref[...], preferred_element_type=jnp.float32)
       + jnp.dot(h_ref[...], wr_ref[...], preferred_element_type=jnp.float32)
       + b_ref[...])
  o_ref[...] = _gelu(y)


def _tc_layer2_body(agg_ref, cnt_ref, h_ref, wl_ref, wr_ref, b_ref,
                    wlin_ref, blin_ref, o_ref):
  agg = agg_ref[0] + agg_ref[1]
  cnt = cnt_ref[0, :, 0:1] + cnt_ref[1, :, 0:1]
  mean = agg / jnp.maximum(cnt, 1.0)
  y = (jnp.dot(mean, wl_ref[...], preferred_element_type=jnp.float32)
       + jnp.dot(h_ref[...], wr_ref[...], preferred_element_type=jnp.float32)
       + b_ref[...])
  g = _gelu(y)
  o_ref[...] = (jnp.dot(g, wlin_ref[...], preferred_element_type=jnp.float32)
                + blin_ref[...])


def _tc_layer(body, agg_parts, cnt_parts, h, mats, out_dim):
  grid = (_N // _BM,)
  in_specs = [
      pl.BlockSpec((_NC, _BM, _D), lambda i: (0, i, 0)),
      pl.BlockSpec((_NC, _BM, _D), lambda i: (0, i, 0)),
      pl.BlockSpec((_BM, _D), lambda i: (i, 0)),
  ]
  args = [agg_parts, cnt_parts, h]
  for m in mats:
    m2 = m if m.ndim == 2 else m.reshape(1, -1)
    in_specs.append(pl.BlockSpec(m2.shape, lambda i: (0, 0)))
    args.append(m2)
  return pl.pallas_call(
      body,
      grid=grid,
      in_specs=in_specs,
      out_specs=pl.BlockSpec((_BM, out_dim), lambda i: (i, 0)),
      out_shape=jax.ShapeDtypeStruct((_N, out_dim), jnp.float32),
  )(*args)


def _prep_edges(ei):
  npad = _EPWP - _EPW
  src = ei[0].reshape(_NW, _EPW)
  dst = ei[1].reshape(_NW, _EPW)
  pad = _N + (jnp.arange(_NW, dtype=jnp.int32)[:, None] * 8
              + jnp.arange(npad, dtype=jnp.int32)[None, :]) % (_NP - _N)
  src = jnp.concatenate([src, jnp.zeros((_NW, npad), jnp.int32)], axis=1)
  dst = jnp.concatenate([dst, pad.astype(jnp.int32)], axis=1)
  return (src.reshape(_NW, _NCHUNK, _CH), dst.reshape(_NW, _NCHUNK, _CH))


def kernel(x, edge_index_0, edge_index_1, W_l0, W_r0, b0, W_l1, W_r1, b1,
           W_lin, b_lin):
  src0, dst0 = _prep_edges(edge_index_0)
  src1, dst1 = _prep_edges(edge_index_1)
  aggp0, cntp0 = _sc_agg(x, src0, dst0)
  h1 = _tc_layer(_tc_layer1_body, aggp0, cntp0, x, (W_l0, W_r0, b0), _D)
  aggp1, cntp1 = _sc_agg(h1, src1, dst1)
  out = _tc_layer(_tc_layer2_body, aggp1, cntp1, h1,
                  (W_l1, W_r1, b1, W_lin, b_lin), _D)
  return out


# speedup vs baseline: 1.4035x; 1.4035x over previous
import functools

import jax
import jax.numpy as jnp
from jax import lax
from jax.experimental import pallas as pl
from jax.experimental.pallas import tpu as pltpu
from jax.experimental.pallas import tpu_sc as plsc

_N = 10000
_D = 128
_E = 320000

_NC = 2
_NS = 16
_NW = _NC * _NS
_EPW = _E // _NW
_CH = 96
_NCHUNK = 105
_EPWP = _NCHUNK * _CH
_NP = 10240
_RPT = _NP // _NS
_ZR = 8


def _sc_agg(h, src, dst):
  mesh = plsc.VectorSubcoreMesh(core_axis_name="c", subcore_axis_name="s")

  @functools.partial(
      pl.kernel,
      out_type=(
          jax.ShapeDtypeStruct((_NC, _NP, _D), jnp.float32),
          jax.ShapeDtypeStruct((_NC, _NP, _D), jnp.float32),
      ),
      mesh=mesh,
      scratch_types=[
          pltpu.VMEM((_CH,), jnp.int32),
          pltpu.VMEM((_CH,), jnp.int32),
          pltpu.VMEM((_CH,), jnp.int32),
          pltpu.VMEM((_CH,), jnp.int32),
          pltpu.VMEM((_NCHUNK, _CH), jnp.int32),
          pltpu.VMEM((_CH, _D), jnp.float32),
          pltpu.VMEM((_CH, _D), jnp.float32),
          pltpu.VMEM((_ZR, _D), jnp.float32),
          pltpu.VMEM_SHARED((_NP, _D), jnp.float32),
          pltpu.SemaphoreType.DMA,
          pltpu.SemaphoreType.DMA,
          pltpu.SemaphoreType.DMA,
          pltpu.SemaphoreType.DMA,
          pltpu.SemaphoreType.DMA,
          pltpu.SemaphoreType.DMA,
          pltpu.SemaphoreType.DMA,
          pltpu.SemaphoreType.DMA,
      ],
  )
  def k(h_hbm, src_hbm, dst_hbm, agg_out, cnt_out,
        sb0, sb1, sb2, sb3, didx_v, rows0_v, rows1_v, zd_v, acc_sp,
        g0, g1, si0, si1, si2, si3, sem_s, sem_z):
    cid = lax.axis_index("c")
    sid = lax.axis_index("s")
    wid = sid * _NC + cid
    sbuf = [sb0, sb1, sb2, sb3]
    sisem = [si0, si1, si2, si3]
    rows = [rows0_v, rows1_v]
    gsem = [g0, g1]

    zero16 = jnp.zeros((16,), jnp.float32)
    one16 = jnp.ones((16,), jnp.float32)

    pltpu.sync_copy(dst_hbm.at[wid], didx_v)

    def fill_zd(t, _):
      zd_v[t // (_D // 16), pl.ds((t % (_D // 16)) * 16, 16)] = zero16
      return 0
    lax.fori_loop(0, _ZR * (_D // 16), fill_zd, 0)

    def zero_own_rows(_unused):
      def zero_slab(z, _):
        r0 = sid * _RPT + z * _ZR
        pltpu.async_copy(zd_v, acc_sp.at[pl.ds(r0, _ZR), :], sem_z)
        return 0
      lax.fori_loop(0, _RPT // _ZR, zero_slab, 0)

      def zero_drain(z, _):
        r0 = sid * _RPT + z * _ZR
        pltpu.make_async_copy(zd_v, acc_sp.at[pl.ds(r0, _ZR), :],
                              sem_z).wait()
        return 0
      lax.fori_loop(0, _RPT // _ZR, zero_drain, 0)

    zero_own_rows(None)
    plsc.subcore_barrier()

    for j in range(4):
      pltpu.async_copy(src_hbm.at[wid, j], sbuf[j], sisem[j])
    pltpu.make_async_copy(src_hbm.at[wid, 0], sbuf[0], sisem[0]).wait()
    pltpu.async_copy(h_hbm.at[sbuf[0]], rows0_v, g0)
    pltpu.make_async_copy(src_hbm.at[wid, 1], sbuf[1], sisem[1]).wait()
    pltpu.async_copy(h_hbm.at[sbuf[1]], rows1_v, g1)

    def quad(gidx, _):
      for b in range(4):
        i = 4 * gidx + b
        r = rows[b % 2]
        pltpu.make_async_copy(h_hbm.at[sbuf[b]], r, gsem[b % 2]).wait()

        @pl.when(i + 4 < _NCHUNK)
        def _():
          pltpu.async_copy(src_hbm.at[wid, i + 4], sbuf[b], sisem[b])
        pltpu.sync_copy(r, acc_sp.at[didx_v.at[i]], add=True)

        @pl.when(i + 2 < _NCHUNK)
        def _():
          b2 = (b + 2) % 4
          pltpu.make_async_copy(src_hbm.at[wid, i + 2], sbuf[b2],
                                sisem[b2]).wait()
          pltpu.async_copy(h_hbm.at[sbuf[b2]], r, gsem[b % 2])
      return 0
    lax.fori_loop(0, (_NCHUNK - 1) // 4, quad, 0)

    pltpu.make_async_copy(h_hbm.at[sbuf[0]], rows0_v, g0).wait()
    pltpu.sync_copy(rows0_v, acc_sp.at[didx_v.at[_NCHUNK - 1]], add=True)

    plsc.subcore_barrier()

    r0 = sid * _RPT
    pltpu.sync_copy(acc_sp.at[pl.ds(r0, _RPT), :],
                    agg_out.at[cid, pl.ds(r0, _RPT), :])

    def fill_ones(t, _):
      rows0_v[t // (_D // 16), pl.ds((t % (_D // 16)) * 16, 16)] = one16
      return 0
    lax.fori_loop(0, _CH * (_D // 16), fill_ones, 0)
    zero_own_rows(None)
    plsc.subcore_barrier()

    def cnt_fire(i, _):
      pltpu.async_copy(rows0_v, acc_sp.at[didx_v.at[i]], sem_s, add=True)
      return 0
    lax.fori_loop(0, _NCHUNK, cnt_fire, 0)

    def cnt_drain(i, _):
      pltpu.make_async_copy(rows0_v, acc_sp.at[didx_v.at[i]], sem_s).wait()
      return 0
    lax.fori_loop(0, _NCHUNK, cnt_drain, 0)

    plsc.subcore_barrier()

    pltpu.sync_copy(acc_sp.at[pl.ds(r0, _RPT), :],
                    cnt_out.at[cid, pl.ds(r0, _RPT), :])

  return k(h, src, dst)


_BM = 1000


def _gelu(y):
  return 0.5 * y * (1.0 + lax.erf(y * 0.7071067811865476))


def _tc_layer1_body(agg_ref, cnt_ref, h_ref, wl_ref, wr_ref, b_ref, o_ref):
  agg = agg_ref[0] + agg_ref[1]
  cnt = cnt_ref[0, :, 0:1] + cnt_ref[1, :, 0:1]
  mean = agg / jnp.maximum(cnt, 1.0)
  y = (jnp.dot(mean, wl_ref[...], preferred_element_type=jnp.float32)
       + jnp.dot(h_ref[...], wr_ref[...], preferred_element_type=jnp.float32)
       + b_ref[...])
  o_ref[...] = _gelu(y)


def _tc_layer2_body(agg_ref, cnt_ref, h_ref, wl_ref, wr_ref, b_ref,
                    wlin_ref, blin_ref, o_ref):
  agg = agg_ref[0] + agg_ref[1]
  cnt = cnt_ref[0, :, 0:1] + cnt_ref[1, :, 0:1]
  mean = agg / jnp.maximum(cnt, 1.0)
  y = (jnp.dot(mean, wl_ref[...], preferred_element_type=jnp.float32)
       + jnp.dot(h_ref[...], wr_ref[...], preferred_element_type=jnp.float32)
       + b_ref[...])
  g = _gelu(y)
  o_ref[...] = (jnp.dot(g, wlin_ref[...], preferred_element_type=jnp.float32)
                + blin_ref[...])


def _tc_layer(body, agg_parts, cnt_parts, h, mats, out_dim):
  grid = (_N // _BM,)
  in_specs = [
      pl.BlockSpec((_NC, _BM, _D), lambda i: (0, i, 0)),
      pl.BlockSpec((_NC, _BM, _D), lambda i: (0, i, 0)),
      pl.BlockSpec((_BM, _D), lambda i: (i, 0)),
  ]
  args = [agg_parts, cnt_parts, h]
  for m in mats:
    m2 = m if m.ndim == 2 else m.reshape(1, -1)
    in_specs.append(pl.BlockSpec(m2.shape, lambda i: (0, 0)))
    args.append(m2)
  return pl.pallas_call(
      body,
      grid=grid,
      in_specs=in_specs,
      out_specs=pl.BlockSpec((_BM, out_dim), lambda i: (i, 0)),
      out_shape=jax.ShapeDtypeStruct((_N, out_dim), jnp.float32),
  )(*args)


def _prep_edges(ei):
  npad = _EPWP - _EPW
  src = ei[0].reshape(_NW, _EPW)
  dst = ei[1].reshape(_NW, _EPW)
  pad = _N + (jnp.arange(_NW, dtype=jnp.int32)[:, None] * 8
              + jnp.arange(npad, dtype=jnp.int32)[None, :]) % (_NP - _N)
  spad = (jnp.arange(_NW, dtype=jnp.int32)[:, None] * 311
          + jnp.arange(npad, dtype=jnp.int32)[None, :] * 97) % _N
  src = jnp.concatenate([src, spad.astype(jnp.int32)], axis=1)
  dst = jnp.concatenate([dst, pad.astype(jnp.int32)], axis=1)
  return (src.reshape(_NW, _NCHUNK, _CH), dst.reshape(_NW, _NCHUNK, _CH))


def kernel(x, edge_index_0, edge_index_1, W_l0, W_r0, b0, W_l1, W_r1, b1,
           W_lin, b_lin):
  src0, dst0 = _prep_edges(edge_index_0)
  src1, dst1 = _prep_edges(edge_index_1)
  aggp0, cntp0 = _sc_agg(x, src0, dst0)
  h1 = _tc_layer(_tc_layer1_body, aggp0, cntp0, x, (W_l0, W_r0, b0), _D)
  aggp1, cntp1 = _sc_agg(h1, src1, dst1)
  out = _tc_layer(_tc_layer2_body, aggp1, cntp1, h1,
                  (W_l1, W_r1, b1, W_lin, b_lin), _D)
  return out


# speedup vs baseline: 1.4767x; 1.0521x over previous
import functools

import jax
import jax.numpy as jnp
from jax import lax
from jax.experimental import pallas as pl
from jax.experimental.pallas import tpu as pltpu
from jax.experimental.pallas import tpu_sc as plsc

_N = 10000
_D = 128
_E = 320000

_NC = 2
_NS = 16
_NW = _NC * _NS
_EPW = _E // _NW
_CH = 80
_NCHUNK = 128
_EPWP = _NCHUNK * _CH
_NP = 10240
_RPT = _NP // _NS
_ZR = 8


def _sc_agg(h, src, dst):
  mesh = plsc.VectorSubcoreMesh(core_axis_name="c", subcore_axis_name="s")

  @functools.partial(
      pl.kernel,
      out_type=(
          jax.ShapeDtypeStruct((_NC, _NP, _D), jnp.float32),
          jax.ShapeDtypeStruct((_NC, _NP, _D), jnp.float32),
      ),
      mesh=mesh,
      scratch_types=[
          pltpu.VMEM((_CH,), jnp.int32),
          pltpu.VMEM((_CH,), jnp.int32),
          pltpu.VMEM((_CH,), jnp.int32),
          pltpu.VMEM((_CH,), jnp.int32),
          pltpu.VMEM((_CH,), jnp.int32),
          pltpu.VMEM((_CH,), jnp.int32),
          pltpu.VMEM((16, _CH), jnp.int32),
          pltpu.VMEM((16, _CH), jnp.int32),
          pltpu.VMEM((_CH, _D), jnp.float32),
          pltpu.VMEM((_CH, _D), jnp.float32),
          pltpu.VMEM((_CH, _D), jnp.float32),
          pltpu.VMEM((_ZR, _D), jnp.float32),
          pltpu.VMEM_SHARED((_NP, _D), jnp.float32),
          pltpu.SemaphoreType.DMA,
          pltpu.SemaphoreType.DMA,
          pltpu.SemaphoreType.DMA,
          pltpu.SemaphoreType.DMA,
          pltpu.SemaphoreType.DMA,
          pltpu.SemaphoreType.DMA,
          pltpu.SemaphoreType.DMA,
          pltpu.SemaphoreType.DMA,
          pltpu.SemaphoreType.DMA,
          pltpu.SemaphoreType.DMA,
          pltpu.SemaphoreType.DMA,
          pltpu.SemaphoreType.DMA,
          pltpu.SemaphoreType.DMA,
          pltpu.SemaphoreType.DMA,
          pltpu.SemaphoreType.DMA,
          pltpu.SemaphoreType.DMA,
      ],
  )
  def k(h_hbm, src_hbm, dst_hbm, agg_out, cnt_out,
        sb0, sb1, sb2, db0, db1, db2, dgA, dgB, r0v, r1v, r2v, zd_v, acc_sp,
        g0, g1, g2, si0, si1, si2, di0, di1, di2, sc0, sc1, sc2,
        dpA, dpB, sem_s, sem_z):
    cid = lax.axis_index("c")
    sid = lax.axis_index("s")
    wid = sid * _NC + cid
    sbuf = [sb0, sb1, sb2]
    dbuf = [db0, db1, db2]
    sisem = [si0, si1, si2]
    disem = [di0, di1, di2]
    rows = [r0v, r1v, r2v]
    gsem = [g0, g1, g2]
    scsem = [sc0, sc1, sc2]

    zero16 = jnp.zeros((16,), jnp.float32)
    one16 = jnp.ones((16,), jnp.float32)

    def fill_zd(t, _):
      zd_v[t // (_D // 16), pl.ds((t % (_D // 16)) * 16, 16)] = zero16
      return 0
    lax.fori_loop(0, _ZR * (_D // 16), fill_zd, 0)

    def zero_own_rows(_unused):
      def zero_slab(z, _):
        r0 = sid * _RPT + z * _ZR
        pltpu.async_copy(zd_v, acc_sp.at[pl.ds(r0, _ZR), :], sem_z)
        return 0
      lax.fori_loop(0, _RPT // _ZR, zero_slab, 0)

      def zero_drain(z, _):
        r0 = sid * _RPT + z * _ZR
        pltpu.make_async_copy(zd_v, acc_sp.at[pl.ds(r0, _ZR), :],
                              sem_z).wait()
        return 0
      lax.fori_loop(0, _RPT // _ZR, zero_drain, 0)

    zero_own_rows(None)
    plsc.subcore_barrier()

    for j in range(3):
      pltpu.async_copy(src_hbm.at[wid, j], sbuf[j], sisem[j])
      pltpu.async_copy(dst_hbm.at[wid, j], dbuf[j], disem[j])
    for j in range(2):
      pltpu.make_async_copy(src_hbm.at[wid, j], sbuf[j], sisem[j]).wait()
      pltpu.async_copy(h_hbm.at[sbuf[j]], rows[j], gsem[j])

    def step(i, b):
      bp = (b + 2) % 3
      pltpu.make_async_copy(h_hbm.at[sbuf[b]], rows[b], gsem[b]).wait()

      @pl.when(i + 3 < _NCHUNK)
      def _():
        pltpu.async_copy(src_hbm.at[wid, i + 3], sbuf[b], sisem[b])
      pltpu.make_async_copy(dst_hbm.at[wid, i], dbuf[b], disem[b]).wait()
      pltpu.async_copy(rows[b], acc_sp.at[dbuf[b]], scsem[b], add=True)

      @pl.when(i > 0)
      def _():
        pltpu.make_async_copy(rows[bp], acc_sp.at[dbuf[bp]],
                              scsem[bp]).wait()

      @pl.when(i + 2 < _NCHUNK)
      def _():
        pltpu.async_copy(dst_hbm.at[wid, i + 2], dbuf[bp], disem[bp])
        pltpu.make_async_copy(src_hbm.at[wid, i + 2], sbuf[bp],
                              sisem[bp]).wait()
        pltpu.async_copy(h_hbm.at[sbuf[bp]], rows[bp], gsem[bp])
      return 0

    def triple(q, _):
      for b in range(3):
        step(3 * q + b, b)
      return 0
    lax.fori_loop(0, (_NCHUNK - 2) // 3, triple, 0)

    step(_NCHUNK - 2, 0)
    step(_NCHUNK - 1, 1)
    pltpu.make_async_copy(rows[1], acc_sp.at[dbuf[1]], scsem[1]).wait()

    plsc.subcore_barrier()

    r0 = sid * _RPT
    pltpu.sync_copy(acc_sp.at[pl.ds(r0, _RPT), :],
                    agg_out.at[cid, pl.ds(r0, _RPT), :])

    def fill_ones(t, _):
      r0v[t // (_D // 16), pl.ds((t % (_D // 16)) * 16, 16)] = one16
      return 0
    lax.fori_loop(0, _CH * (_D // 16), fill_ones, 0)
    zero_own_rows(None)
    plsc.subcore_barrier()

    dgrp = [dgA, dgB]
    dgsem = [dpA, dpB]
    ngrp = _NCHUNK // 16
    pltpu.async_copy(dst_hbm.at[wid, pl.ds(0, 16)], dgA, dpA)
    pltpu.async_copy(dst_hbm.at[wid, pl.ds(16, 16)], dgB, dpB)

    def cnt_grp(gi, p):
      buf = dgrp[p]
      sem = dgsem[p]
      pltpu.make_async_copy(
          dst_hbm.at[wid, pl.ds(16 * gi, 16)], buf, sem).wait()
      for b in range(16):
        pltpu.async_copy(r0v, acc_sp.at[buf.at[b]], sem_s, add=True)
      for b in range(16):
        pltpu.make_async_copy(r0v, acc_sp.at[buf.at[b]], sem_s).wait()

      @pl.when(gi + 2 < ngrp)
      def _():
        pltpu.async_copy(dst_hbm.at[wid, pl.ds(16 * (gi + 2), 16)], buf, sem)

    def cnt_pair(q, _):
      cnt_grp(2 * q, 0)
      cnt_grp(2 * q + 1, 1)
      return 0
    lax.fori_loop(0, ngrp // 2, cnt_pair, 0)

    plsc.subcore_barrier()

    pltpu.sync_copy(acc_sp.at[pl.ds(r0, _RPT), :],
                    cnt_out.at[cid, pl.ds(r0, _RPT), :])

  return k(h, src, dst)


_BM = 1000


def _gelu(y):
  return 0.5 * y * (1.0 + lax.erf(y * 0.7071067811865476))


def _tc_layer1_body(agg_ref, cnt_ref, h_ref, wl_ref, wr_ref, b_ref, o_ref):
  agg = agg_ref[0] + agg_ref[1]
  cnt = cnt_ref[0, :, 0:1] + cnt_ref[1, :, 0:1]
  mean = agg / jnp.maximum(cnt, 1.0)
  y = (jnp.dot(mean, wl_ref[...], preferred_element_type=jnp.float32)
       + jnp.dot(h_ref[...], wr_ref[...], preferred_element_type=jnp.float32)
       + b_ref[...])
  o_ref[...] = _gelu(y)


def _tc_layer2_body(agg_ref, cnt_ref, h_ref, wl_ref, wr_ref, b_ref,
                    wlin_ref, blin_ref, o_ref):
  agg = agg_ref[0] + agg_ref[1]
  cnt = cnt_ref[0, :, 0:1] + cnt_ref[1, :, 0:1]
  mean = agg / jnp.maximum(cnt, 1.0)
  y = (jnp.dot(mean, wl_ref[...], preferred_element_type=jnp.float32)
       + jnp.dot(h_ref[...], wr_ref[...], preferred_element_type=jnp.float32)
       + b_ref[...])
  g = _gelu(y)
  o_ref[...] = (jnp.dot(g, wlin_ref[...], preferred_element_type=jnp.float32)
                + blin_ref[...])


def _tc_layer(body, agg_parts, cnt_parts, h, mats, out_dim):
  grid = (_N // _BM,)
  in_specs = [
      pl.BlockSpec((_NC, _BM, _D), lambda i: (0, i, 0)),
      pl.BlockSpec((_NC, _BM, _D), lambda i: (0, i, 0)),
      pl.BlockSpec((_BM, _D), lambda i: (i, 0)),
  ]
  args = [agg_parts, cnt_parts, h]
  for m in mats:
    m2 = m if m.ndim == 2 else m.reshape(1, -1)
    in_specs.append(pl.BlockSpec(m2.shape, lambda i: (0, 0)))
    args.append(m2)
  return pl.pallas_call(
      body,
      grid=grid,
      in_specs=in_specs,
      out_specs=pl.BlockSpec((_BM, out_dim), lambda i: (i, 0)),
      out_shape=jax.ShapeDtypeStruct((_N, out_dim), jnp.float32),
  )(*args)


def _prep_edges(ei):
  npad = _EPWP - _EPW
  src = ei[0].reshape(_NW, _EPW)
  dst = ei[1].reshape(_NW, _EPW)
  spad = (jnp.arange(_NW, dtype=jnp.int32)[:, None] * 311
          + jnp.arange(npad, dtype=jnp.int32)[None, :] * 97) % _N
  dpad = _N + (jnp.arange(_NW, dtype=jnp.int32)[:, None] * 8
               + jnp.arange(npad, dtype=jnp.int32)[None, :]) % (_NP - _N)
  src = jnp.concatenate([src, spad.astype(jnp.int32)], axis=1)
  dst = jnp.concatenate([dst, dpad.astype(jnp.int32)], axis=1)
  return (src.reshape(_NW, _NCHUNK, _CH), dst.reshape(_NW, _NCHUNK, _CH))


def kernel(x, edge_index_0, edge_index_1, W_l0, W_r0, b0, W_l1, W_r1, b1,
           W_lin, b_lin):
  src0, dst0 = _prep_edges(edge_index_0)
  src1, dst1 = _prep_edges(edge_index_1)
  aggp0, cntp0 = _sc_agg(x, src0, dst0)
  h1 = _tc_layer(_tc_layer1_body, aggp0, cntp0, x, (W_l0, W_r0, b0), _D)
  aggp1, cntp1 = _sc_agg(h1, src1, dst1)
  out = _tc_layer(_tc_layer2_body, aggp1, cntp1, h1,
                  (W_l1, W_r1, b1, W_lin, b_lin), _D)
  return out


# speedup vs baseline: 1.4950x; 1.0124x over previous
import functools

import jax
import jax.numpy as jnp
from jax import lax
from jax.experimental import pallas as pl
from jax.experimental.pallas import tpu as pltpu
from jax.experimental.pallas import tpu_sc as plsc

_N = 10000
_D = 128
_E = 320000

_NC = 2
_NS = 16
_NW = _NC * _NS
_EPW = _E // _NW
_CH = 80
_NCHUNK = 128
_EPWP = _NCHUNK * _CH
_NP = 10240
_RPT = _NP // _NS
_ZR = 8


def _sc_agg(h, src, dst):
  mesh = plsc.VectorSubcoreMesh(core_axis_name="c", subcore_axis_name="s")

  @functools.partial(
      pl.kernel,
      out_type=(
          jax.ShapeDtypeStruct((_NC, _NP, _D), jnp.float32),
          jax.ShapeDtypeStruct((_NC, _NP, _D), jnp.float32),
      ),
      mesh=mesh,
      scratch_types=[
          pltpu.VMEM((_CH,), jnp.int32),
          pltpu.VMEM((_CH,), jnp.int32),
          pltpu.VMEM((_CH,), jnp.int32),
          pltpu.VMEM((_CH,), jnp.int32),
          pltpu.VMEM((_CH,), jnp.int32),
          pltpu.VMEM((_CH,), jnp.int32),
          pltpu.VMEM((16, _CH), jnp.int32),
          pltpu.VMEM((16, _CH), jnp.int32),
          pltpu.VMEM((_CH, _D), jnp.float32),
          pltpu.VMEM((_CH, _D), jnp.float32),
          pltpu.VMEM((_CH, _D), jnp.float32),
          pltpu.VMEM((_ZR, _D), jnp.float32),
          pltpu.VMEM_SHARED((_NP, _D), jnp.float32),
          pltpu.SemaphoreType.DMA,
          pltpu.SemaphoreType.DMA,
          pltpu.SemaphoreType.DMA,
          pltpu.SemaphoreType.DMA,
          pltpu.SemaphoreType.DMA,
          pltpu.SemaphoreType.DMA,
          pltpu.SemaphoreType.DMA,
          pltpu.SemaphoreType.DMA,
          pltpu.SemaphoreType.DMA,
          pltpu.SemaphoreType.DMA,
          pltpu.SemaphoreType.DMA,
          pltpu.SemaphoreType.DMA,
          pltpu.SemaphoreType.DMA,
          pltpu.SemaphoreType.DMA,
          pltpu.SemaphoreType.DMA,
          pltpu.SemaphoreType.DMA,
      ],
  )
  def k(h_hbm, src_hbm, dst_hbm, agg_out, cnt_out,
        sb0, sb1, sb2, db0, db1, db2, dgA, dgB, r0v, r1v, r2v, zd_v, acc_sp,
        g0, g1, g2, si0, si1, si2, di0, di1, di2, sc0, sc1, sc2,
        dpA, dpB, sem_s, sem_z):
    cid = lax.axis_index("c")
    sid = lax.axis_index("s")
    wid = sid * _NC + cid
    sbuf = [sb0, sb1, sb2]
    dbuf = [db0, db1, db2]
    sisem = [si0, si1, si2]
    disem = [di0, di1, di2]
    rows = [r0v, r1v, r2v]
    gsem = [g0, g1, g2]
    scsem = [sc0, sc1, sc2]

    zero16 = jnp.zeros((16,), jnp.float32)
    one16 = jnp.ones((16,), jnp.float32)

    def fill_zd(t, _):
      zd_v[t // (_D // 16), pl.ds((t % (_D // 16)) * 16, 16)] = zero16
      return 0
    lax.fori_loop(0, _ZR * (_D // 16), fill_zd, 0)

    def zero_own_rows(_unused):
      def zero_slab(z, _):
        r0 = sid * _RPT + z * _ZR
        pltpu.async_copy(zd_v, acc_sp.at[pl.ds(r0, _ZR), :], sem_z)
        return 0
      lax.fori_loop(0, _RPT // _ZR, zero_slab, 0)

      def zero_drain(z, _):
        r0 = sid * _RPT + z * _ZR
        pltpu.make_async_copy(zd_v, acc_sp.at[pl.ds(r0, _ZR), :],
                              sem_z).wait()
        return 0
      lax.fori_loop(0, _RPT // _ZR, zero_drain, 0)

    for j in range(3):
      pltpu.async_copy(src_hbm.at[wid, j], sbuf[j], sisem[j])
      pltpu.async_copy(dst_hbm.at[wid, j], dbuf[j], disem[j])
    for j in range(2):
      pltpu.make_async_copy(src_hbm.at[wid, j], sbuf[j], sisem[j]).wait()
      pltpu.async_copy(h_hbm.at[sbuf[j]], rows[j], gsem[j])
    zero_own_rows(None)
    plsc.subcore_barrier()

    def step(i, b):
      bp = (b + 2) % 3
      pltpu.make_async_copy(h_hbm.at[sbuf[b]], rows[b], gsem[b]).wait()

      @pl.when(i > 0)
      def _():
        pltpu.make_async_copy(rows[bp], acc_sp.at[dbuf[bp]],
                              scsem[bp]).wait()

      @pl.when(i + 2 < _NCHUNK)
      def _():
        pltpu.make_async_copy(src_hbm.at[wid, i + 2], sbuf[bp],
                              sisem[bp]).wait()
        pltpu.async_copy(h_hbm.at[sbuf[bp]], rows[bp], gsem[bp])

      @pl.when(i + 3 < _NCHUNK)
      def _():
        pltpu.async_copy(src_hbm.at[wid, i + 3], sbuf[b], sisem[b])
      pltpu.make_async_copy(dst_hbm.at[wid, i], dbuf[b], disem[b]).wait()
      pltpu.async_copy(rows[b], acc_sp.at[dbuf[b]], scsem[b], add=True)

      @pl.when(i + 2 < _NCHUNK)
      def _():
        pltpu.async_copy(dst_hbm.at[wid, i + 2], dbuf[bp], disem[bp])
      return 0

    def triple(q, _):
      for b in range(3):
        step(3 * q + b, b)
      return 0
    lax.fori_loop(0, (_NCHUNK - 2) // 3, triple, 0)

    step(_NCHUNK - 2, 0)
    step(_NCHUNK - 1, 1)
    pltpu.make_async_copy(rows[1], acc_sp.at[dbuf[1]], scsem[1]).wait()

    plsc.subcore_barrier()

    r0 = sid * _RPT
    pltpu.sync_copy(acc_sp.at[pl.ds(r0, _RPT), :],
                    agg_out.at[cid, pl.ds(r0, _RPT), :])

    def fill_ones(t, _):
      r0v[t // (_D // 16), pl.ds((t % (_D // 16)) * 16, 16)] = one16
      return 0
    lax.fori_loop(0, _CH * (_D // 16), fill_ones, 0)
    zero_own_rows(None)
    plsc.subcore_barrier()

    dgrp = [dgA, dgB]
    dgsem = [dpA, dpB]
    ngrp = _NCHUNK // 16
    pltpu.async_copy(dst_hbm.at[wid, pl.ds(0, 16)], dgA, dpA)
    pltpu.async_copy(dst_hbm.at[wid, pl.ds(16, 16)], dgB, dpB)

    def cnt_grp(gi, p):
      buf = dgrp[p]
      sem = dgsem[p]
      pltpu.make_async_copy(
          dst_hbm.at[wid, pl.ds(16 * gi, 16)], buf, sem).wait()
      for b in range(16):
        pltpu.async_copy(r0v, acc_sp.at[buf.at[b]], sem_s, add=True)
      for b in range(16):
        pltpu.make_async_copy(r0v, acc_sp.at[buf.at[b]], sem_s).wait()

      @pl.when(gi + 2 < ngrp)
      def _():
        pltpu.async_copy(dst_hbm.at[wid, pl.ds(16 * (gi + 2), 16)], buf, sem)

    def cnt_pair(q, _):
      cnt_grp(2 * q, 0)
      cnt_grp(2 * q + 1, 1)
      return 0
    lax.fori_loop(0, ngrp // 2, cnt_pair, 0)

    plsc.subcore_barrier()

    pltpu.sync_copy(acc_sp.at[pl.ds(r0, _RPT), :],
                    cnt_out.at[cid, pl.ds(r0, _RPT), :])

  return k(h, src, dst)


_BM = 1000


def _gelu(y):
  return 0.5 * y * (1.0 + lax.erf(y * 0.7071067811865476))


def _tc_layer1_body(agg_ref, cnt_ref, h_ref, wl_ref, wr_ref, b_ref, o_ref):
  agg = agg_ref[0] + agg_ref[1]
  cnt = cnt_ref[0, :, 0:1] + cnt_ref[1, :, 0:1]
  mean = agg / jnp.maximum(cnt, 1.0)
  y = (jnp.dot(mean, wl_ref[...], preferred_element_type=jnp.float32)
       + jnp.dot(h_ref[...], wr_ref[...], preferred_element_type=jnp.float32)
       + b_ref[...])
  o_ref[...] = _gelu(y)


def _tc_layer2_body(agg_ref, cnt_ref, h_ref, wl_ref, wr_ref, b_ref,
                    wlin_ref, blin_ref, o_ref):
  agg = agg_ref[0] + agg_ref[1]
  cnt = cnt_ref[0, :, 0:1] + cnt_ref[1, :, 0:1]
  mean = agg / jnp.maximum(cnt, 1.0)
  y = (jnp.dot(mean, wl_ref[...], preferred_element_type=jnp.float32)
       + jnp.dot(h_ref[...], wr_ref[...], preferred_element_type=jnp.float32)
       + b_ref[...])
  g = _gelu(y)
  o_ref[...] = (jnp.dot(g, wlin_ref[...], preferred_element_type=jnp.float32)
                + blin_ref[...])


def _tc_layer(body, agg_parts, cnt_parts, h, mats, out_dim):
  grid = (_N // _BM,)
  in_specs = [
      pl.BlockSpec((_NC, _BM, _D), lambda i: (0, i, 0)),
      pl.BlockSpec((_NC, _BM, _D), lambda i: (0, i, 0)),
      pl.BlockSpec((_BM, _D), lambda i: (i, 0)),
  ]
  args = [agg_parts, cnt_parts, h]
  for m in mats:
    m2 = m if m.ndim == 2 else m.reshape(1, -1)
    in_specs.append(pl.BlockSpec(m2.shape, lambda i: (0, 0)))
    args.append(m2)
  return pl.pallas_call(
      body,
      grid=grid,
      in_specs=in_specs,
      out_specs=pl.BlockSpec((_BM, out_dim), lambda i: (i, 0)),
      out_shape=jax.ShapeDtypeStruct((_N, out_dim), jnp.float32),
  )(*args)


def _prep_edges(ei):
  npad = _EPWP - _EPW
  src = ei[0].reshape(_NW, _EPW)
  dst = ei[1].reshape(_NW, _EPW)
  spad = (jnp.arange(_NW, dtype=jnp.int32)[:, None] * 311
          + jnp.arange(npad, dtype=jnp.int32)[None, :] * 97) % _N
  dpad = _N + (jnp.arange(_NW, dtype=jnp.int32)[:, None] * 8
               + jnp.arange(npad, dtype=jnp.int32)[None, :]) % (_NP - _N)
  src = jnp.concatenate([src, spad.astype(jnp.int32)], axis=1)
  dst = jnp.concatenate([dst, dpad.astype(jnp.int32)], axis=1)
  return (src.reshape(_NW, _NCHUNK, _CH), dst.reshape(_NW, _NCHUNK, _CH))


def kernel(x, edge_index_0, edge_index_1, W_l0, W_r0, b0, W_l1, W_r1, b1,
           W_lin, b_lin):
  src0, dst0 = _prep_edges(edge_index_0)
  src1, dst1 = _prep_edges(edge_index_1)
  aggp0, cntp0 = _sc_agg(x, src0, dst0)
  h1 = _tc_layer(_tc_layer1_body, aggp0, cntp0, x, (W_l0, W_r0, b0), _D)
  aggp1, cntp1 = _sc_agg(h1, src1, dst1)
  out = _tc_layer(_tc_layer2_body, aggp1, cntp1, h1,
                  (W_l1, W_r1, b1, W_lin, b_lin), _D)
  return out
